# Initial kernel scaffold; baseline (speedup 1.0000x reference)
import jax
import jax.numpy as jnp
from jax.experimental import pallas as pl


def kernel(x, weights, anchors, detector_input_ids):
    raise NotImplementedError("write your pallas kernel here")


# speedup vs baseline: 7.7917x; 7.7917x over previous
import functools

import jax
import jax.numpy as jnp
from jax import lax
from jax.experimental import pallas as pl
from jax.experimental.pallas import tpu as pltpu
from jax.experimental.pallas import tpu_sc as plsc

B = 8
S = 20
T = B * S
NDET = 1024
NANCH = 16
NOUT = 64
NLOOKUP = NDET * NANCH

NC = 2
NS = 16
L = 16

TPC = T // NC
DPS = NDET // NS
TG = TPC // L


def _body(x_hbm, w_hbm, anch_hbm, ids_hbm,
          out_hbm, lut_hbm, mind_hbm, amin_hbm,
          x_v, w_v, anch_v, ids_v, lut_v, mind_v, amin_v, acc_v, tidx_v,
          acc_sh):
    cid = lax.axis_index("c")
    sid = lax.axis_index("s")
    t0 = cid * TPC
    d0 = sid * DPS

    pltpu.sync_copy(x_hbm.at[pl.ds(t0, TPC), :], x_v)
    pltpu.sync_copy(w_hbm.at[pl.ds(d0 * NANCH, DPS * NANCH), :], w_v)
    pltpu.sync_copy(anch_hbm.at[pl.ds(d0, DPS), :], anch_v)
    pltpu.sync_copy(ids_hbm.at[pl.ds(d0, DPS)], ids_v)

    iota = lax.iota(jnp.int32, L)
    for g in range(TPC // L):
        tidx_v[pl.ds(g * L, L)] = iota + g * L

    def d_body(d, carry):
        di = jnp.full((L,), d, jnp.int32)
        fi = plsc.load_gather(ids_v, [di])
        gd16 = (d0 + d) * NANCH
        arow = anch_v[d, :]
        anchs = [jnp.full((L,), arow[a]) for a in range(NANCH)]
        for tg in range(TG):
            ti = iota + tg * L
            xi = plsc.load_gather(x_v, [ti, fi])
            best = jnp.abs(xi - anchs[0])
            besta = jnp.zeros((L,), jnp.int32)
            for a in range(1, NANCH):
                dl = jnp.abs(xi - anchs[a])
                m = dl < best
                besta = jnp.where(m, a, besta)
                best = jnp.where(m, dl, best)
            plsc.store_scatter(mind_v, [ti, di], best)
            plsc.store_scatter(amin_v, [ti, di], besta)
            plsc.store_scatter(lut_v, [ti, di], gd16 + besta)
        return carry

    lax.fori_loop(0, DPS, d_body, 0)

    pltpu.sync_copy(lut_v, lut_hbm.at[pl.ds(t0, TPC), pl.ds(d0, DPS)])
    pltpu.sync_copy(mind_v, mind_hbm.at[pl.ds(t0, TPC), pl.ds(d0, DPS)])
    pltpu.sync_copy(amin_v, amin_hbm.at[pl.ds(t0, TPC), pl.ds(d0, DPS)])

    zero = jnp.zeros((L,), jnp.float32)

    def t_body(t, carry):
        def dd_body(dg, accs):
            a0, a1, a2, a3 = accs
            amv = amin_v[t, pl.ds(dg * L, L)]
            cv = 1.0 - mind_v[t, pl.ds(dg * L, L)]
            for u in range(L):
                r = (dg * L + u) * NANCH + amv[u]
                c = cv[u]
                a0 = a0 + c * w_v[r, pl.ds(0, L)]
                a1 = a1 + c * w_v[r, pl.ds(L, L)]
                a2 = a2 + c * w_v[r, pl.ds(2 * L, L)]
                a3 = a3 + c * w_v[r, pl.ds(3 * L, L)]
            return (a0, a1, a2, a3)

        a0, a1, a2, a3 = lax.fori_loop(0, DPS // L, dd_body,
                                       (zero, zero, zero, zero))
        acc_v[t, pl.ds(0, L)] = a0
        acc_v[t, pl.ds(L, L)] = a1
        acc_v[t, pl.ds(2 * L, L)] = a2
        acc_v[t, pl.ds(3 * L, L)] = a3
        return carry

    lax.fori_loop(0, TPC, t_body, 0)

    plsc.subcore_barrier()

    @pl.when(sid == 0)
    def _():
        pltpu.sync_copy(acc_v, acc_sh)

    plsc.subcore_barrier()

    @pl.when(sid != 0)
    def _():
        pltpu.sync_copy(acc_v, acc_sh.at[tidx_v], add=True)

    plsc.subcore_barrier()

    @pl.when(sid == 0)
    def _():
        pltpu.sync_copy(acc_sh, out_hbm.at[pl.ds(t0, TPC), :])


_lut_sc = functools.partial(
    pl.kernel,
    out_type=(
        jax.ShapeDtypeStruct((T, NOUT), jnp.float32),
        jax.ShapeDtypeStruct((T, NDET), jnp.int32),
        jax.ShapeDtypeStruct((T, NDET), jnp.float32),
        jax.ShapeDtypeStruct((T, NDET), jnp.int32),
    ),
    mesh=plsc.VectorSubcoreMesh(core_axis_name="c", subcore_axis_name="s",
                                num_cores=NC, num_subcores=NS),
    compiler_params=pltpu.CompilerParams(use_tc_tiling_on_sc=False,
                                         needs_layout_passes=False),
    scratch_types=[
        pltpu.VMEM((TPC, NOUT), jnp.float32),
        pltpu.VMEM((DPS * NANCH, NOUT), jnp.float32),
        pltpu.VMEM((DPS, NANCH), jnp.float32),
        pltpu.VMEM((DPS,), jnp.int32),
        pltpu.VMEM((TPC, DPS), jnp.int32),
        pltpu.VMEM((TPC, DPS), jnp.float32),
        pltpu.VMEM((TPC, DPS), jnp.int32),
        pltpu.VMEM((TPC, NOUT), jnp.float32),
        pltpu.VMEM((TPC,), jnp.int32),
        pltpu.VMEM_SHARED((TPC, NOUT), jnp.float32),
    ],
)(_body)


@jax.jit
def kernel(x, weights, anchors, detector_input_ids):
    xb, xs, _ = x.shape
    x2 = x.reshape(T, NOUT)
    out, lut, mind, amin = _lut_sc(x2, weights, anchors, detector_input_ids)
    return (out.reshape(xb, xs, NOUT),
            lut.reshape(xb, xs, NDET),
            mind.reshape(xb, xs, NDET),
            amin.reshape(xb, xs, NDET))
